# baseline (speedup 1.0000x reference)
import functools

import jax
import jax.numpy as jnp
from jax import lax
from jax.experimental import pallas as pl
from jax.experimental.pallas import tpu as pltpu
from jax.experimental.pallas import tpu_sc as plsc

N = 10000
E = 320000
NC = 2
NS = 16
EPW = E // (NC * NS)
C = 40
NCHUNK = EPW // C
RA = 624
TAIL = N - NS * RA
MW = 128


def _make_sc_agg(dout):
    tw = 3 * dout
    nh = dout // 16
    P = 2 * C
    NPAIR = EPW // P
    mesh = plsc.VectorSubcoreMesh(core_axis_name="c", subcore_axis_name="s")

    @functools.partial(
        pl.kernel, mesh=mesh,
        compiler_params=pltpu.CompilerParams(use_tc_tiling_on_sc=False),
        out_type=jax.ShapeDtypeStruct((NC, N, dout), jnp.float32),
        scratch_types=[
            pltpu.VMEM((C,), jnp.int32),
            pltpu.VMEM((C,), jnp.int32),
            pltpu.VMEM((C,), jnp.int32),
            pltpu.VMEM((C,), jnp.int32),
            pltpu.VMEM((256,), jnp.float32),
            pltpu.VMEM((2, C, tw), jnp.float32),
            pltpu.VMEM((P, dout), jnp.float32),
            pltpu.VMEM_SHARED((N, dout), jnp.float32),
            pltpu.SemaphoreType.DMA,
            pltpu.SemaphoreType.DMA,
            pltpu.SemaphoreType.DMA,
            pltpu.SemaphoreType.DMA,
            pltpu.SemaphoreType.DMA,
            pltpu.SemaphoreType.DMA,
            pltpu.SemaphoreType.DMA,
            pltpu.SemaphoreType.DMA,
            pltpu.SemaphoreType.DMA,
            pltpu.SemaphoreType.DMA,
        ],
    )
    def k(y_hbm, src_hbm, dst_hbm, abc_hbm, zero_hbm, out_hbm,
          sidx0, sidx1, didx0, didx1, abcv, rows, msg, agg,
          ss0, ss1, sa0, sa1, sg0, sg1, sd0, sd1, sc0, sc1):
        cid = lax.axis_index("c")
        sid = lax.axis_index("s")
        base = (cid * NS + sid) * EPW
        sidx = (sidx0, sidx1)
        didx = (didx0, didx1)
        ssem = (ss0, ss1)
        asem = (sa0, sa1)
        gsem = (sg0, sg1)
        dsem = (sd0, sd1)
        csem = (sc0, sc1)

        def start_sidx(i, b):
            off = base + jnp.minimum(i, NCHUNK - 1) * C
            pltpu.make_async_copy(src_hbm.at[pl.ds(off, C)],
                                  sidx[b], ssem[b]).start()

        def wait_sidx(b):
            pltpu.make_async_copy(src_hbm.at[pl.ds(0, C)],
                                  sidx[b], ssem[b]).wait()

        def start_abc(i, b):
            off = base + jnp.minimum(i, NCHUNK - 1) * C
            pltpu.make_async_copy(abc_hbm.at[pl.ds(off * 3, C * 3)],
                                  abcv.at[pl.ds(b * 128, C * 3)], asem[b]).start()

        def wait_abc(b):
            pltpu.make_async_copy(abc_hbm.at[pl.ds(0, C * 3)],
                                  abcv.at[pl.ds(b * 128, C * 3)], asem[b]).wait()

        def start_didx(i, b):
            off = base + jnp.minimum(i, NCHUNK - 1) * C
            pltpu.make_async_copy(dst_hbm.at[pl.ds(off, C)],
                                  didx[b], dsem[b]).start()

        def wait_didx(b):
            pltpu.make_async_copy(dst_hbm.at[pl.ds(0, C)],
                                  didx[b], dsem[b]).wait()

        def start_scatter(b):
            pltpu.async_copy(msg.at[pl.ds(b * C, C)], agg.at[didx[b]],
                             csem[b], add=True)

        def wait_scatter(b):
            pltpu.make_async_copy(msg.at[pl.ds(b * C, C)], agg.at[didx[b]],
                                  csem[b]).wait()

        def start_gather(b):
            pltpu.make_async_copy(y_hbm.at[sidx[b]], rows.at[b],
                                  gsem[b]).start()

        def wait_gather(b):
            pltpu.make_async_copy(y_hbm.at[sidx[b]], rows.at[b],
                                  gsem[b]).wait()

        def compute(b):
            rb = rows.at[b]

            @plsc.parallel_loop(0, C // 8)
            def _grp(g):
                abg = abcv[pl.ds(b * 128 + g * 24, 16)]
                bcg = abcv[pl.ds(b * 128 + g * 24 + 8, 16)]
                for j in range(8):
                    aw = jnp.take(abg, jnp.full((16,), j, jnp.int32), mode="wrap")
                    bw = jnp.take(abg, jnp.full((16,), 8 + j, jnp.int32), mode="wrap")
                    cw = jnp.take(bcg, jnp.full((16,), 8 + j, jnp.int32), mode="wrap")
                    e = g * 8 + j
                    for h in range(nh):
                        r0 = rb[e, pl.ds(h * 16, 16)]
                        r1 = rb[e, pl.ds(dout + h * 16, 16)]
                        r2 = rb[e, pl.ds(2 * dout + h * 16, 16)]
                        msg[b * C + e, pl.ds(h * 16, 16)] = aw * r0 + bw * r1 + cw * r2

        pltpu.sync_copy(zero_hbm.at[pl.ds(sid * RA, RA)],
                        agg.at[pl.ds(sid * RA, RA)])

        @pl.when(sid == NS - 1)
        def _ztail():
            pltpu.sync_copy(zero_hbm.at[pl.ds(NS * RA, TAIL)],
                            agg.at[pl.ds(NS * RA, TAIL)])

        @pl.loop(0, P)
        def _zpad(e):
            for h in range(nh):
                msg[e, pl.ds(h * 16, 16)] = jnp.zeros((16,), jnp.float32)

        plsc.subcore_barrier()

        start_sidx(0, 0)
        start_sidx(1, 1)
        start_abc(0, 0)
        start_abc(1, 1)
        start_didx(0, 0)
        start_didx(1, 1)
        wait_sidx(0)
        start_gather(0)
        wait_didx(0)
        wait_didx(1)
        start_scatter(0)
        start_scatter(1)

        @pl.loop(0, NPAIR)
        def _pair(p):
            i = p * 2
            wait_sidx(1)
            start_gather(1)
            wait_gather(0)
            wait_abc(0)
            start_sidx(i + 2, 0)
            wait_scatter(0)
            start_didx(i, 0)
            compute(0)
            start_abc(i + 2, 0)
            wait_didx(0)
            start_scatter(0)
            wait_sidx(0)
            start_gather(0)
            wait_gather(1)
            wait_abc(1)
            start_sidx(i + 3, 1)
            wait_scatter(1)
            start_didx(i + 1, 1)
            compute(1)
            start_abc(i + 3, 1)
            wait_didx(1)
            start_scatter(1)

        wait_sidx(1)
        wait_abc(0)
        wait_abc(1)
        wait_gather(0)
        wait_scatter(0)
        wait_scatter(1)

        plsc.subcore_barrier()
        pltpu.sync_copy(agg.at[pl.ds(sid * RA, RA)],
                        out_hbm.at[cid, pl.ds(sid * RA, RA)])

        @pl.when(sid == NS - 1)
        def _otail():
            pltpu.sync_copy(agg.at[pl.ds(NS * RA, TAIL)],
                            out_hbm.at[cid, pl.ds(NS * RA, TAIL)])

    return k


def _edge_scalars(theta, g, distance):
    t2 = theta.reshape(2500, 128)
    g2 = g.reshape(2500, 128)
    d2 = distance.reshape(2500, 128)

    def body(t_ref, g_ref, d_ref, a_ref, b_ref, c_ref):
        a = jnp.exp(-d_ref[...]) * jnp.cos(g_ref[...])
        a_ref[...] = a
        b_ref[...] = a * jnp.cos(t_ref[...])
        c_ref[...] = a * jnp.sin(t_ref[...])

    out = jax.ShapeDtypeStruct((2500, 128), jnp.float32)
    a2, b2, c2 = pl.pallas_call(body, out_shape=(out, out, out))(t2, g2, d2)
    return a2.reshape(E), b2.reshape(E), c2.reshape(E)


def _mm_first(h0, wall, tw, dout):
    din = h0.shape[1]
    wtot = wall.shape[1]
    blk = 1000

    def body(x_ref, w_ref, y1_ref, y2_ref):
        y = jnp.dot(x_ref[...], w_ref[...], preferred_element_type=jnp.float32)
        y1_ref[...] = y[:, :tw]
        y2_ref[...] = y[:, tw:]

    return pl.pallas_call(
        body,
        grid=(N // blk,),
        in_specs=[pl.BlockSpec((blk, din), lambda i: (i, 0)),
                  pl.BlockSpec((din, wtot), lambda i: (0, 0))],
        out_specs=[pl.BlockSpec((blk, tw), lambda i: (i, 0)),
                   pl.BlockSpec((blk, dout), lambda i: (i, 0))],
        out_shape=(jax.ShapeDtypeStruct((N, tw), jnp.float32),
                   jax.ShapeDtypeStruct((N, dout), jnp.float32)),
    )(h0, wall)


def _mm_combine(agg_a, agg_b, s_prev, h_prev, wall, tw, dout):
    din = s_prev.shape[1]
    wtot = wall.shape[1]
    blk = 1000
    residual = h_prev is not None

    def body(*refs):
        if residual:
            aa, ab, sp, hp, w, y1, y2, ho = refs
            pre = hp[...] + aa[...] + ab[...] + sp[...]
        else:
            aa, ab, sp, w, y1, y2, ho = refs
            pre = aa[...] + ab[...] + sp[...]
        h = jnp.maximum(pre, 0.0)
        ho[...] = h
        y = jnp.dot(h, w[...], preferred_element_type=jnp.float32)
        y1[...] = y[:, :tw]
        y2[...] = y[:, tw:]

    nspec = pl.BlockSpec((blk, din), lambda i: (i, 0))
    in_specs = [nspec, nspec, nspec] + ([nspec] if residual else []) + [
        pl.BlockSpec((din, wtot), lambda i: (0, 0))]
    args = (agg_a, agg_b, s_prev) + ((h_prev,) if residual else ()) + (wall,)
    return pl.pallas_call(
        body,
        grid=(N // blk,),
        in_specs=in_specs,
        out_specs=[pl.BlockSpec((blk, tw), lambda i: (i, 0)),
                   pl.BlockSpec((blk, dout), lambda i: (i, 0)),
                   pl.BlockSpec((blk, din), lambda i: (i, 0))],
        out_shape=(jax.ShapeDtypeStruct((N, tw), jnp.float32),
                   jax.ShapeDtypeStruct((N, dout), jnp.float32),
                   jax.ShapeDtypeStruct((N, din), jnp.float32)),
    )(*args)


def _final_relu(agg_a, agg_b, s_prev):
    din = s_prev.shape[1]
    blk = 1000

    def body(aa, ab, sp, o):
        o[...] = jnp.maximum(aa[...] + ab[...] + sp[...], 0.0)

    nspec = pl.BlockSpec((blk, din), lambda i: (i, 0))
    return pl.pallas_call(
        body,
        grid=(N // blk,),
        in_specs=[nspec, nspec, nspec],
        out_specs=nspec,
        out_shape=jax.ShapeDtypeStruct((N, din), jnp.float32),
    )(agg_a, agg_b, s_prev)


_sc_agg_96 = _make_sc_agg(96)
_sc_agg_64 = _make_sc_agg(64)


def _pack_wall(w0, wc, ws, wself):
    return jnp.concatenate([w0, wc, ws, wself], axis=1)


def kernel(pos, x, edge_index, theta, g, distance,
           W0_0, Wc_0, Ws_0, Wsl_0,
           W0_1, Wc_1, Ws_1, Wsl_1,
           W0_2, Wc_2, Ws_2, Wsl_2):
    h0 = jnp.concatenate([pos, x], axis=1)
    src = edge_index[0].astype(jnp.int32)
    dst = edge_index[1].astype(jnp.int32)

    a, b, c = _edge_scalars(theta, g, distance)
    abc = jnp.concatenate([a.reshape(-1, 8), b.reshape(-1, 8),
                           c.reshape(-1, 8)], axis=1).reshape(E * 3)

    z96 = jnp.zeros((N, 96), jnp.float32)
    z64 = jnp.zeros((N, 64), jnp.float32)

    wall0 = _pack_wall(W0_0, Wc_0, Ws_0, Wsl_0)
    wall1 = _pack_wall(W0_1, Wc_1, Ws_1, Wsl_1)
    wall2 = _pack_wall(W0_2, Wc_2, Ws_2, Wsl_2)

    ycat0, s0 = _mm_first(h0, wall0, 288, 96)
    agg0 = _sc_agg_96(ycat0, src, dst, abc, z96)
    ycat1, s1, h1 = _mm_combine(agg0[0], agg0[1], s0, None, wall1, 288, 96)
    agg1 = _sc_agg_96(ycat1, src, dst, abc, z96)
    ycat2, s2, _h2 = _mm_combine(agg1[0], agg1[1], s1, h1, wall2, 192, 64)
    agg2 = _sc_agg_64(ycat2, src, dst, abc, z64)
    return _final_relu(agg2[0], agg2[1], s2)


# speedup vs baseline: 1.0645x; 1.0645x over previous
import functools

import jax
import jax.numpy as jnp
from jax import lax
from jax.experimental import pallas as pl
from jax.experimental.pallas import tpu as pltpu
from jax.experimental.pallas import tpu_sc as plsc

N = 10000
E = 320000
NC = 2
NS = 16
EPW = E // (NC * NS)
C = 40
NCHUNK = EPW // C
RA = 624
TAIL = N - NS * RA
MW = 128


def _make_sc_agg(dout):
    tw = 3 * dout
    nh = dout // 16
    P = 2 * C
    NPAIR = EPW // P
    mesh = plsc.VectorSubcoreMesh(core_axis_name="c", subcore_axis_name="s")

    @functools.partial(
        pl.kernel, mesh=mesh,
        compiler_params=pltpu.CompilerParams(use_tc_tiling_on_sc=False),
        out_type=jax.ShapeDtypeStruct((NC, N, dout), jnp.float32),
        scratch_types=[
            pltpu.VMEM((C,), jnp.int32),
            pltpu.VMEM((C,), jnp.int32),
            pltpu.VMEM((C,), jnp.int32),
            pltpu.VMEM((C,), jnp.int32),
            pltpu.VMEM((96,), jnp.float32),
            pltpu.VMEM((96,), jnp.float32),
            pltpu.VMEM((96,), jnp.float32),
            pltpu.VMEM((2, C, tw), jnp.float32),
            pltpu.VMEM((P, dout), jnp.float32),
            pltpu.VMEM_SHARED((N, dout), jnp.float32),
            pltpu.SemaphoreType.DMA,
            pltpu.SemaphoreType.DMA,
            pltpu.SemaphoreType.DMA,
            pltpu.SemaphoreType.DMA,
            pltpu.SemaphoreType.DMA,
            pltpu.SemaphoreType.DMA,
            pltpu.SemaphoreType.DMA,
            pltpu.SemaphoreType.DMA,
            pltpu.SemaphoreType.DMA,
            pltpu.SemaphoreType.DMA,
        ],
    )
    def k(y_hbm, src_hbm, dst_hbm, a_hbm, b_hbm, c_hbm, zero_hbm, out_hbm,
          sidx0, sidx1, didx0, didx1, av, bv, cv, rows, msg, agg,
          ss0, ss1, sa0, sa1, sg0, sg1, sd0, sd1, sc0, sc1):
        cid = lax.axis_index("c")
        sid = lax.axis_index("s")
        base = (cid * NS + sid) * EPW
        sidx = (sidx0, sidx1)
        didx = (didx0, didx1)
        ssem = (ss0, ss1)
        asem = (sa0, sa1)
        gsem = (sg0, sg1)
        dsem = (sd0, sd1)
        csem = (sc0, sc1)

        def start_sidx(i, b):
            off = base + jnp.minimum(i, NCHUNK - 1) * C
            pltpu.make_async_copy(src_hbm.at[pl.ds(off, C)],
                                  sidx[b], ssem[b]).start()

        def wait_sidx(b):
            pltpu.make_async_copy(src_hbm.at[pl.ds(0, C)],
                                  sidx[b], ssem[b]).wait()

        def start_abc(i, b):
            off = base + jnp.minimum(i, NCHUNK - 1) * C
            for hbm, buf in ((a_hbm, av), (b_hbm, bv), (c_hbm, cv)):
                pltpu.make_async_copy(hbm.at[pl.ds(off, C)],
                                      buf.at[pl.ds(b * 48, C)], asem[b]).start()

        def wait_abc(b):
            for hbm, buf in ((a_hbm, av), (b_hbm, bv), (c_hbm, cv)):
                pltpu.make_async_copy(hbm.at[pl.ds(0, C)],
                                      buf.at[pl.ds(b * 48, C)], asem[b]).wait()

        def start_didx(i, b):
            off = base + jnp.minimum(i, NCHUNK - 1) * C
            pltpu.make_async_copy(dst_hbm.at[pl.ds(off, C)],
                                  didx[b], dsem[b]).start()

        def wait_didx(b):
            pltpu.make_async_copy(dst_hbm.at[pl.ds(0, C)],
                                  didx[b], dsem[b]).wait()

        def start_scatter(b):
            pltpu.async_copy(msg.at[pl.ds(b * C, C)], agg.at[didx[b]],
                             csem[b], add=True)

        def wait_scatter(b):
            pltpu.make_async_copy(msg.at[pl.ds(b * C, C)], agg.at[didx[b]],
                                  csem[b]).wait()

        def start_gather(b):
            pltpu.make_async_copy(y_hbm.at[sidx[b]], rows.at[b],
                                  gsem[b]).start()

        def wait_gather(b):
            pltpu.make_async_copy(y_hbm.at[sidx[b]], rows.at[b],
                                  gsem[b]).wait()

        def compute(b):
            rb = rows.at[b]

            @plsc.parallel_loop(0, C // 8)
            def _grp(g):
                ag = av[pl.ds(b * 48 + g * 8, 16)]
                bg = bv[pl.ds(b * 48 + g * 8, 16)]
                cg = cv[pl.ds(b * 48 + g * 8, 16)]
                for j in range(8):
                    lane = jnp.full((16,), j, jnp.int32)
                    aw = jnp.take(ag, lane, mode="wrap")
                    bw = jnp.take(bg, lane, mode="wrap")
                    cw = jnp.take(cg, lane, mode="wrap")
                    e = g * 8 + j
                    for h in range(nh):
                        r0 = rb[e, pl.ds(h * 16, 16)]
                        r1 = rb[e, pl.ds(dout + h * 16, 16)]
                        r2 = rb[e, pl.ds(2 * dout + h * 16, 16)]
                        msg[b * C + e, pl.ds(h * 16, 16)] = aw * r0 + bw * r1 + cw * r2

        pltpu.sync_copy(zero_hbm.at[pl.ds(sid * RA, RA)],
                        agg.at[pl.ds(sid * RA, RA)])

        @pl.when(sid == NS - 1)
        def _ztail():
            pltpu.sync_copy(zero_hbm.at[pl.ds(NS * RA, TAIL)],
                            agg.at[pl.ds(NS * RA, TAIL)])

        @pl.loop(0, P)
        def _zpad(e):
            for h in range(nh):
                msg[e, pl.ds(h * 16, 16)] = jnp.zeros((16,), jnp.float32)

        plsc.subcore_barrier()

        start_sidx(0, 0)
        start_sidx(1, 1)
        start_abc(0, 0)
        start_abc(1, 1)
        start_didx(0, 0)
        start_didx(1, 1)
        wait_sidx(0)
        start_gather(0)
        wait_didx(0)
        wait_didx(1)
        start_scatter(0)
        start_scatter(1)

        @pl.loop(0, NPAIR)
        def _pair(p):
            i = p * 2
            wait_sidx(1)
            start_gather(1)
            wait_gather(0)
            wait_abc(0)
            start_sidx(i + 2, 0)
            wait_scatter(0)
            start_didx(i, 0)
            compute(0)
            start_abc(i + 2, 0)
            wait_didx(0)
            start_scatter(0)
            wait_sidx(0)
            start_gather(0)
            wait_gather(1)
            wait_abc(1)
            start_sidx(i + 3, 1)
            wait_scatter(1)
            start_didx(i + 1, 1)
            compute(1)
            start_abc(i + 3, 1)
            wait_didx(1)
            start_scatter(1)

        wait_sidx(1)
        wait_abc(0)
        wait_abc(1)
        wait_gather(0)
        wait_scatter(0)
        wait_scatter(1)

        plsc.subcore_barrier()
        pltpu.sync_copy(agg.at[pl.ds(sid * RA, RA)],
                        out_hbm.at[cid, pl.ds(sid * RA, RA)])

        @pl.when(sid == NS - 1)
        def _otail():
            pltpu.sync_copy(agg.at[pl.ds(NS * RA, TAIL)],
                            out_hbm.at[cid, pl.ds(NS * RA, TAIL)])

    return k


def _edge_scalars(theta, g, distance):
    t2 = theta.reshape(2500, 128)
    g2 = g.reshape(2500, 128)
    d2 = distance.reshape(2500, 128)

    def body(t_ref, g_ref, d_ref, a_ref, b_ref, c_ref):
        a = jnp.exp(-d_ref[...]) * jnp.cos(g_ref[...])
        a_ref[...] = a
        b_ref[...] = a * jnp.cos(t_ref[...])
        c_ref[...] = a * jnp.sin(t_ref[...])

    out = jax.ShapeDtypeStruct((2500, 128), jnp.float32)
    a2, b2, c2 = pl.pallas_call(body, out_shape=(out, out, out))(t2, g2, d2)
    return a2.reshape(E), b2.reshape(E), c2.reshape(E)


def _mm_first(h0, wall, tw, dout):
    din = h0.shape[1]
    wtot = wall.shape[1]
    blk = 1000

    def body(x_ref, w_ref, y1_ref, y2_ref):
        y = jnp.dot(x_ref[...], w_ref[...], preferred_element_type=jnp.float32)
        y1_ref[...] = y[:, :tw]
        y2_ref[...] = y[:, tw:]

    return pl.pallas_call(
        body,
        grid=(N // blk,),
        in_specs=[pl.BlockSpec((blk, din), lambda i: (i, 0)),
                  pl.BlockSpec((din, wtot), lambda i: (0, 0))],
        out_specs=[pl.BlockSpec((blk, tw), lambda i: (i, 0)),
                   pl.BlockSpec((blk, dout), lambda i: (i, 0))],
        out_shape=(jax.ShapeDtypeStruct((N, tw), jnp.float32),
                   jax.ShapeDtypeStruct((N, dout), jnp.float32)),
    )(h0, wall)


def _mm_combine(agg_a, agg_b, s_prev, h_prev, wall, tw, dout):
    din = s_prev.shape[1]
    wtot = wall.shape[1]
    blk = 1000
    residual = h_prev is not None

    def body(*refs):
        if residual:
            aa, ab, sp, hp, w, y1, y2, ho = refs
            pre = hp[...] + aa[...] + ab[...] + sp[...]
        else:
            aa, ab, sp, w, y1, y2, ho = refs
            pre = aa[...] + ab[...] + sp[...]
        h = jnp.maximum(pre, 0.0)
        ho[...] = h
        y = jnp.dot(h, w[...], preferred_element_type=jnp.float32)
        y1[...] = y[:, :tw]
        y2[...] = y[:, tw:]

    nspec = pl.BlockSpec((blk, din), lambda i: (i, 0))
    in_specs = [nspec, nspec, nspec] + ([nspec] if residual else []) + [
        pl.BlockSpec((din, wtot), lambda i: (0, 0))]
    args = (agg_a, agg_b, s_prev) + ((h_prev,) if residual else ()) + (wall,)
    return pl.pallas_call(
        body,
        grid=(N // blk,),
        in_specs=in_specs,
        out_specs=[pl.BlockSpec((blk, tw), lambda i: (i, 0)),
                   pl.BlockSpec((blk, dout), lambda i: (i, 0)),
                   pl.BlockSpec((blk, din), lambda i: (i, 0))],
        out_shape=(jax.ShapeDtypeStruct((N, tw), jnp.float32),
                   jax.ShapeDtypeStruct((N, dout), jnp.float32),
                   jax.ShapeDtypeStruct((N, din), jnp.float32)),
    )(*args)


def _final_relu(agg_a, agg_b, s_prev):
    din = s_prev.shape[1]
    blk = 1000

    def body(aa, ab, sp, o):
        o[...] = jnp.maximum(aa[...] + ab[...] + sp[...], 0.0)

    nspec = pl.BlockSpec((blk, din), lambda i: (i, 0))
    return pl.pallas_call(
        body,
        grid=(N // blk,),
        in_specs=[nspec, nspec, nspec],
        out_specs=nspec,
        out_shape=jax.ShapeDtypeStruct((N, din), jnp.float32),
    )(agg_a, agg_b, s_prev)


_sc_agg_96 = _make_sc_agg(96)
_sc_agg_64 = _make_sc_agg(64)


def _pack_wall(w0, wc, ws, wself):
    return jnp.concatenate([w0, wc, ws, wself], axis=1)


def kernel(pos, x, edge_index, theta, g, distance,
           W0_0, Wc_0, Ws_0, Wsl_0,
           W0_1, Wc_1, Ws_1, Wsl_1,
           W0_2, Wc_2, Ws_2, Wsl_2):
    h0 = jnp.concatenate([pos, x], axis=1)
    src = edge_index[0].astype(jnp.int32)
    dst = edge_index[1].astype(jnp.int32)

    a, b, c = _edge_scalars(theta, g, distance)

    z96 = jnp.zeros((N, 96), jnp.float32)
    z64 = jnp.zeros((N, 64), jnp.float32)

    wall0 = _pack_wall(W0_0, Wc_0, Ws_0, Wsl_0)
    wall1 = _pack_wall(W0_1, Wc_1, Ws_1, Wsl_1)
    wall2 = _pack_wall(W0_2, Wc_2, Ws_2, Wsl_2)

    ycat0, s0 = _mm_first(h0, wall0, 288, 96)
    agg0 = _sc_agg_96(ycat0, src, dst, a, b, c, z96)
    ycat1, s1, h1 = _mm_combine(agg0[0], agg0[1], s0, None, wall1, 288, 96)
    agg1 = _sc_agg_96(ycat1, src, dst, a, b, c, z96)
    ycat2, s2, _h2 = _mm_combine(agg1[0], agg1[1], s1, h1, wall2, 192, 64)
    agg2 = _sc_agg_64(ycat2, src, dst, a, b, c, z64)
    return _final_relu(agg2[0], agg2[1], s2)
